# baseline (speedup 1.0000x reference)
import functools

import jax
import jax.numpy as jnp
from jax import lax
from jax.experimental import pallas as pl
from jax.experimental.pallas import tpu as pltpu
from jax.experimental.pallas import tpu_sc as plsc

N = 10000
E = 320000
D = 128
H = 8
NP = 10240
RW = 128
NWORK = 32
RPW = 80
NROWS = NWORK * RPW
EP = NROWS * RW
TSLAB = NP // 16

_mesh = plsc.VectorSubcoreMesh(core_axis_name="c", subcore_axis_name="s")
_sc_params = pltpu.CompilerParams(
    use_tc_tiling_on_sc=False, needs_layout_passes=False)


@functools.partial(
    pl.kernel,
    mesh=_mesh,
    compiler_params=_sc_params,
    out_type=jax.ShapeDtypeStruct((2, NP), jnp.float32),
    scratch_types=[
        pltpu.VMEM((RPW, RW), jnp.int32),
        pltpu.VMEM((RPW, RW), jnp.float32),
        pltpu.VMEM_SHARED((NP,), jnp.float32),
        pltpu.SemaphoreType.DMA,
    ],
)
def _deg_kernel(dst_hbm, w_hbm, zeros_hbm, deg_out, didx, wv, deg_sh, sem):
    c = lax.axis_index("c")
    s = lax.axis_index("s")
    wid = s * 2 + c
    pltpu.sync_copy(zeros_hbm.at[pl.ds(s * TSLAB, TSLAB)],
                    deg_sh.at[pl.ds(s * TSLAB, TSLAB)])
    pltpu.sync_copy(dst_hbm.at[pl.ds(wid * RPW, RPW)], didx)
    pltpu.sync_copy(w_hbm.at[pl.ds(wid * RPW, RPW)], wv)
    plsc.subcore_barrier()

    @pl.loop(0, RPW)
    def _(j):
        pltpu.async_copy(wv.at[j], deg_sh.at[didx.at[j]], sem, add=True)

    @pl.loop(0, RPW)
    def _(j):
        pltpu.make_async_copy(wv.at[j], deg_sh.at[didx.at[j]], sem).wait()

    plsc.subcore_barrier()
    pltpu.sync_copy(deg_sh.at[pl.ds(s * TSLAB, TSLAB)],
                    deg_out.at[c].at[pl.ds(s * TSLAB, TSLAB)])


@functools.partial(
    pl.kernel,
    mesh=_mesh,
    compiler_params=_sc_params,
    out_type=jax.ShapeDtypeStruct((2, NP, H), jnp.float32),
    scratch_types=[
        pltpu.VMEM((RPW, RW), jnp.int32),
        pltpu.VMEM((RPW, RW), jnp.int32),
        pltpu.VMEM((RPW, RW), jnp.float32),
        pltpu.VMEM((NP,), jnp.float32),
        pltpu.VMEM((RW,), jnp.float32),
        pltpu.VMEM((RW, H), jnp.float32),
        pltpu.VMEM((RW, H), jnp.float32),
        pltpu.VMEM((RW, H), jnp.float32),
        pltpu.VMEM((RW, H), jnp.float32),
        pltpu.VMEM_SHARED((N, H), jnp.float32),
        pltpu.VMEM_SHARED((NP, H), jnp.float32),
        pltpu.SemaphoreType.DMA,
        pltpu.SemaphoreType.DMA,
        pltpu.SemaphoreType.DMA,
        pltpu.SemaphoreType.DMA,
    ],
)
def _agg_kernel(h_hbm, dis_hbm, src_hbm, dst_hbm, w_hbm, zeros_hbm, out_hbm,
                sidx, didx, wv, dis_t, norm_b, rows0, rows1, msg0, msg1,
                h_sh, out_sh, gsem0, gsem1, ssem0, ssem1):
    c = lax.axis_index("c")
    s = lax.axis_index("s")
    wid = s * 2 + c
    pltpu.sync_copy(zeros_hbm.at[pl.ds(s * TSLAB, TSLAB)],
                    out_sh.at[pl.ds(s * TSLAB, TSLAB)])

    @pl.when(s < 15)
    def _():
        pltpu.sync_copy(h_hbm.at[pl.ds(s * TSLAB, TSLAB)],
                        h_sh.at[pl.ds(s * TSLAB, TSLAB)])

    @pl.when(s == 15)
    def _():
        pltpu.sync_copy(h_hbm.at[pl.ds(15 * TSLAB, N - 15 * TSLAB)],
                        h_sh.at[pl.ds(15 * TSLAB, N - 15 * TSLAB)])

    pltpu.sync_copy(src_hbm.at[pl.ds(wid * RPW, RPW)], sidx)
    pltpu.sync_copy(dst_hbm.at[pl.ds(wid * RPW, RPW)], didx)
    pltpu.sync_copy(w_hbm.at[pl.ds(wid * RPW, RPW)], wv)
    pltpu.sync_copy(dis_hbm, dis_t)
    plsc.subcore_barrier()

    lanes = jnp.arange(16, dtype=jnp.int32)
    rep = lanes // 8
    col = lanes - 8 * rep

    def compute_row(j, rows, msg):
        for v in range(RW // 16):
            s16 = sidx[j, pl.ds(16 * v, 16)]
            d16 = didx[j, pl.ds(16 * v, 16)]
            w16 = wv[j, pl.ds(16 * v, 16)]
            nrm = plsc.load_gather(dis_t, [s16]) * w16 * \
                plsc.load_gather(dis_t, [d16])
            norm_b[pl.ds(16 * v, 16)] = nrm

        @pl.loop(0, RW // 2, unroll=8)
        def _(t):
            rowp = 2 * t + rep
            vals = plsc.load_gather(rows, [rowp, col])
            nrm16 = plsc.load_gather(norm_b, [rowp])
            plsc.store_scatter(msg, [rowp, col], vals * nrm16)

    def gather_wait(j, rows, gsem):
        pltpu.make_async_copy(h_sh.at[sidx.at[j]], rows, gsem).wait()

    def scatter_wait(j, msg, ssem):
        pltpu.make_async_copy(msg, out_sh.at[didx.at[j]], ssem).wait()

    pltpu.async_copy(h_sh.at[sidx.at[0]], rows0, gsem0)

    @pl.loop(0, RPW // 2)
    def _(p):
        j0 = 2 * p
        j1 = 2 * p + 1
        pltpu.async_copy(h_sh.at[sidx.at[j1]], rows1, gsem1)
        gather_wait(j0, rows0, gsem0)

        @pl.when(p > 0)
        def _():
            scatter_wait(j0 - 2, msg0, ssem0)

        compute_row(j0, rows0, msg0)
        pltpu.async_copy(msg0, out_sh.at[didx.at[j0]], ssem0, add=True)

        @pl.when(p < RPW // 2 - 1)
        def _():
            pltpu.async_copy(h_sh.at[sidx.at[j0 + 2]], rows0, gsem0)

        gather_wait(j1, rows1, gsem1)

        @pl.when(p > 0)
        def _():
            scatter_wait(j1 - 2, msg1, ssem1)

        compute_row(j1, rows1, msg1)
        pltpu.async_copy(msg1, out_sh.at[didx.at[j1]], ssem1, add=True)

    scatter_wait(RPW - 2, msg0, ssem0)
    scatter_wait(RPW - 1, msg1, ssem1)
    plsc.subcore_barrier()
    pltpu.sync_copy(out_sh.at[pl.ds(s * TSLAB, TSLAB)],
                    out_hbm.at[c].at[pl.ds(s * TSLAB, TSLAB)])


def _mm1_body(x_ref, w_ref, deg_ref, o_ref, dis_ref):
    o_ref[...] = jnp.dot(x_ref[...], w_ref[...],
                         preferred_element_type=jnp.float32)
    dis_ref[...] = lax.rsqrt(deg_ref[0] + deg_ref[1] + 1.0)


def _layer1_body(agg_ref, h_ref, dis_ref, b_ref, w_ref, o_ref):
    z = (agg_ref[0, :N] + agg_ref[1, :N]
         + h_ref[...] * dis_ref[:N] * dis_ref[:N] + b_ref[...])
    z = jnp.maximum(z, 0.0)
    o_ref[...] = jnp.dot(z, w_ref[...], preferred_element_type=jnp.float32)


def _layer2_body(agg_ref, h_ref, dis_ref, b_ref, o_ref):
    z = (agg_ref[0, :N] + agg_ref[1, :N]
         + h_ref[...] * dis_ref[:N] * dis_ref[:N] + b_ref[...])
    m = jnp.max(z, axis=1, keepdims=True)
    e = jnp.exp(z - m)
    o_ref[...] = (z - m) - jnp.log(jnp.sum(e, axis=1, keepdims=True))


def kernel(x, edge_index, edge_attr, W1, b1, W2, b2):
    ei = edge_index.astype(jnp.int32)
    src = jnp.pad(ei[0], (0, EP - E)).reshape(NROWS, RW)
    dst = jnp.pad(ei[1], (0, EP - E)).reshape(NROWS, RW)
    w2d = jnp.pad(edge_attr, (0, EP - E)).reshape(NROWS, RW)
    zeros_nh = jnp.zeros((NP, H), jnp.float32)
    zeros_n = jnp.zeros((NP,), jnp.float32)

    deg2 = _deg_kernel(dst, w2d, zeros_n)

    h1, dis2d = pl.pallas_call(
        _mm1_body,
        out_shape=(jax.ShapeDtypeStruct((N, H), jnp.float32),
                   jax.ShapeDtypeStruct((NP // 128, 128), jnp.float32)),
    )(x, W1, deg2.reshape(2, NP // 128, 128))
    dis = dis2d.reshape(NP)
    dis_col = dis2d.reshape(NP, 1)

    agg1 = _agg_kernel(h1, dis, src, dst, w2d, zeros_nh)

    h2 = pl.pallas_call(
        _layer1_body,
        out_shape=jax.ShapeDtypeStruct((N, H), jnp.float32),
    )(agg1, h1, dis_col, b1.reshape(1, H), W2)

    agg2 = _agg_kernel(h2, dis, src, dst, w2d, zeros_nh)

    out = pl.pallas_call(
        _layer2_body,
        out_shape=jax.ShapeDtypeStruct((N, H), jnp.float32),
    )(agg2, h2, dis_col, b2.reshape(1, H))
    return out


# speedup vs baseline: 1.0419x; 1.0419x over previous
import functools

import jax
import jax.numpy as jnp
from jax import lax
from jax.experimental import pallas as pl
from jax.experimental.pallas import tpu as pltpu
from jax.experimental.pallas import tpu_sc as plsc

N = 10000
E = 320000
D = 128
H = 8
NP = 10240
RW = 128
NWORK = 32
RPW = 80
NROWS = NWORK * RPW
EP = NROWS * RW
TSLAB = NP // 16

_mesh = plsc.VectorSubcoreMesh(core_axis_name="c", subcore_axis_name="s")
_sc_params = pltpu.CompilerParams(
    use_tc_tiling_on_sc=False, needs_layout_passes=False)


@functools.partial(
    pl.kernel,
    mesh=_mesh,
    compiler_params=_sc_params,
    out_type=jax.ShapeDtypeStruct((2, NP), jnp.float32),
    scratch_types=[
        pltpu.VMEM((RPW, RW), jnp.int32),
        pltpu.VMEM((RPW, RW), jnp.float32),
        pltpu.VMEM_SHARED((NP,), jnp.float32),
        pltpu.SemaphoreType.DMA,
    ],
)
def _deg_kernel(dst_hbm, w_hbm, zeros_hbm, deg_out, didx, wv, deg_sh, sem):
    c = lax.axis_index("c")
    s = lax.axis_index("s")
    wid = s * 2 + c
    pltpu.sync_copy(zeros_hbm.at[pl.ds(s * TSLAB, TSLAB)],
                    deg_sh.at[pl.ds(s * TSLAB, TSLAB)])
    pltpu.sync_copy(dst_hbm.at[pl.ds(wid * RPW, RPW)], didx)
    pltpu.sync_copy(w_hbm.at[pl.ds(wid * RPW, RPW)], wv)
    plsc.subcore_barrier()

    @pl.loop(0, RPW)
    def _(j):
        pltpu.async_copy(wv.at[j], deg_sh.at[didx.at[j]], sem, add=True)

    @pl.loop(0, RPW)
    def _(j):
        pltpu.make_async_copy(wv.at[j], deg_sh.at[didx.at[j]], sem).wait()

    plsc.subcore_barrier()
    pltpu.sync_copy(deg_sh.at[pl.ds(s * TSLAB, TSLAB)],
                    deg_out.at[c].at[pl.ds(s * TSLAB, TSLAB)])


@functools.partial(
    pl.kernel,
    mesh=_mesh,
    compiler_params=_sc_params,
    out_type=jax.ShapeDtypeStruct((2, NP, H), jnp.float32),
    scratch_types=[
        pltpu.VMEM((RPW, RW), jnp.int32),
        pltpu.VMEM((RPW, RW), jnp.int32),
        pltpu.VMEM((RPW, RW), jnp.float32),
        pltpu.VMEM((NP,), jnp.float32),
        pltpu.VMEM((RW,), jnp.float32),
        pltpu.VMEM((RW, H), jnp.float32),
        pltpu.VMEM((RW, H), jnp.float32),
        pltpu.VMEM((RW, H), jnp.float32),
        pltpu.VMEM((RW, H), jnp.float32),
        pltpu.VMEM_SHARED((N, H), jnp.float32),
        pltpu.VMEM_SHARED((NP, H), jnp.float32),
        pltpu.SemaphoreType.DMA,
        pltpu.SemaphoreType.DMA,
        pltpu.SemaphoreType.DMA,
        pltpu.SemaphoreType.DMA,
    ],
)
def _agg_kernel(h_hbm, dis_hbm, src_hbm, dst_hbm, w_hbm, zeros_hbm, out_hbm,
                sidx, didx, wv, dis_t, norm_b, rows0, rows1, msg0, msg1,
                h_sh, out_sh, gsem0, gsem1, ssem0, ssem1):
    c = lax.axis_index("c")
    s = lax.axis_index("s")
    wid = s * 2 + c
    pltpu.sync_copy(zeros_hbm.at[pl.ds(s * TSLAB, TSLAB)],
                    out_sh.at[pl.ds(s * TSLAB, TSLAB)])

    @pl.when(s < 15)
    def _():
        pltpu.sync_copy(h_hbm.at[pl.ds(s * TSLAB, TSLAB)],
                        h_sh.at[pl.ds(s * TSLAB, TSLAB)])

    @pl.when(s == 15)
    def _():
        pltpu.sync_copy(h_hbm.at[pl.ds(15 * TSLAB, N - 15 * TSLAB)],
                        h_sh.at[pl.ds(15 * TSLAB, N - 15 * TSLAB)])

    pltpu.sync_copy(src_hbm.at[pl.ds(wid * RPW, RPW)], sidx)
    pltpu.sync_copy(dst_hbm.at[pl.ds(wid * RPW, RPW)], didx)
    pltpu.sync_copy(w_hbm.at[pl.ds(wid * RPW, RPW)], wv)
    pltpu.sync_copy(dis_hbm, dis_t)
    plsc.subcore_barrier()

    lanes = jnp.arange(16, dtype=jnp.int32)
    rep = lanes // 8
    col = lanes - 8 * rep

    def compute_row(j, rows, msg):
        for v in range(RW // 16):
            s16 = sidx[j, pl.ds(16 * v, 16)]
            d16 = didx[j, pl.ds(16 * v, 16)]
            w16 = wv[j, pl.ds(16 * v, 16)]
            nrm = plsc.load_gather(dis_t, [s16]) * w16 * \
                plsc.load_gather(dis_t, [d16])
            norm_b[pl.ds(16 * v, 16)] = nrm

        @pl.loop(0, RW // 2, unroll=8)
        def _(t):
            rowp = 2 * t + rep
            vals = plsc.load_gather(rows, [rowp, col])
            nrm16 = plsc.load_gather(norm_b, [rowp])
            plsc.store_scatter(msg, [rowp, col], vals * nrm16)

    def gather_wait(j, rows, gsem):
        pltpu.make_async_copy(h_sh.at[sidx.at[j]], rows, gsem).wait()

    def scatter_wait(j, msg, ssem):
        pltpu.make_async_copy(msg, out_sh.at[didx.at[j]], ssem).wait()

    pltpu.async_copy(h_sh.at[sidx.at[0]], rows0, gsem0)

    @pl.loop(0, RPW // 2)
    def _(p):
        j0 = 2 * p
        j1 = 2 * p + 1
        pltpu.async_copy(h_sh.at[sidx.at[j1]], rows1, gsem1)
        gather_wait(j0, rows0, gsem0)

        @pl.when(p > 0)
        def _():
            scatter_wait(j0 - 2, msg0, ssem0)

        compute_row(j0, rows0, msg0)
        pltpu.async_copy(msg0, out_sh.at[didx.at[j0]], ssem0, add=True)

        @pl.when(p < RPW // 2 - 1)
        def _():
            pltpu.async_copy(h_sh.at[sidx.at[j0 + 2]], rows0, gsem0)

        gather_wait(j1, rows1, gsem1)

        @pl.when(p > 0)
        def _():
            scatter_wait(j1 - 2, msg1, ssem1)

        compute_row(j1, rows1, msg1)
        pltpu.async_copy(msg1, out_sh.at[didx.at[j1]], ssem1, add=True)

    scatter_wait(RPW - 2, msg0, ssem0)
    scatter_wait(RPW - 1, msg1, ssem1)
    plsc.subcore_barrier()
    pltpu.sync_copy(out_sh.at[pl.ds(s * TSLAB, TSLAB)],
                    out_hbm.at[c].at[pl.ds(s * TSLAB, TSLAB)])


def _mm1_body(x_ref, w_ref, o_ref):
    o_ref[...] = jnp.dot(x_ref[...], w_ref[...],
                         preferred_element_type=jnp.float32)


def _dis_body(deg_ref, o_ref):
    o_ref[...] = lax.rsqrt(deg_ref[0] + deg_ref[1] + 1.0)


def _layer1_body(agg_ref, h_ref, dis_ref, b_ref, w_ref, o_ref):
    z = (agg_ref[0, :N] + agg_ref[1, :N]
         + h_ref[...] * dis_ref[:N] * dis_ref[:N] + b_ref[...])
    z = jnp.maximum(z, 0.0)
    o_ref[...] = jnp.dot(z, w_ref[...], preferred_element_type=jnp.float32)


def _layer2_body(agg_ref, h_ref, dis_ref, b_ref, o_ref):
    z = (agg_ref[0, :N] + agg_ref[1, :N]
         + h_ref[...] * dis_ref[:N] * dis_ref[:N] + b_ref[...])
    m = jnp.max(z, axis=1, keepdims=True)
    e = jnp.exp(z - m)
    o_ref[...] = (z - m) - jnp.log(jnp.sum(e, axis=1, keepdims=True))


def kernel(x, edge_index, edge_attr, W1, b1, W2, b2):
    ei = edge_index.astype(jnp.int32)
    src = jnp.pad(ei[0], (0, EP - E)).reshape(NROWS, RW)
    dst = jnp.pad(ei[1], (0, EP - E)).reshape(NROWS, RW)
    w2d = jnp.pad(edge_attr, (0, EP - E)).reshape(NROWS, RW)
    zeros_nh = jnp.zeros((NP, H), jnp.float32)
    zeros_n = jnp.zeros((NP,), jnp.float32)

    deg2 = _deg_kernel(dst, w2d, zeros_n)

    h1 = pl.pallas_call(
        _mm1_body,
        out_shape=jax.ShapeDtypeStruct((N, H), jnp.float32),
    )(x, W1)

    dis2d = pl.pallas_call(
        _dis_body,
        out_shape=jax.ShapeDtypeStruct((NP // 128, 128), jnp.float32),
    )(deg2.reshape(2, NP // 128, 128))
    dis = dis2d.reshape(NP)
    dis_col = dis2d.reshape(NP, 1)

    agg1 = _agg_kernel(h1, dis, src, dst, w2d, zeros_nh)

    h2 = pl.pallas_call(
        _layer1_body,
        out_shape=jax.ShapeDtypeStruct((N, H), jnp.float32),
    )(agg1, h1, dis_col, b1.reshape(1, H), W2)

    agg2 = _agg_kernel(h2, dis, src, dst, w2d, zeros_nh)

    out = pl.pallas_call(
        _layer2_body,
        out_shape=jax.ShapeDtypeStruct((N, H), jnp.float32),
    )(agg2, h2, dis_col, b2.reshape(1, H))
    return out


# speedup vs baseline: 1.3243x; 1.2711x over previous
import functools

import jax
import jax.numpy as jnp
from jax import lax
from jax.experimental import pallas as pl
from jax.experimental.pallas import tpu as pltpu
from jax.experimental.pallas import tpu_sc as plsc

N = 10000
E = 320000
D = 128
H = 8
NP = 10240
RW = 128
NWORK = 32
RPW = 80
NROWS = NWORK * RPW
EP = NROWS * RW
TSLAB = NP // 16

_mesh = plsc.VectorSubcoreMesh(core_axis_name="c", subcore_axis_name="s")
_sc_params = pltpu.CompilerParams(
    use_tc_tiling_on_sc=False, needs_layout_passes=False)


@functools.partial(
    pl.kernel,
    mesh=_mesh,
    compiler_params=_sc_params,
    out_type=jax.ShapeDtypeStruct((2, NP), jnp.float32),
    scratch_types=[
        pltpu.VMEM((RPW, RW), jnp.int32),
        pltpu.VMEM((RPW, RW), jnp.float32),
        pltpu.VMEM_SHARED((NP,), jnp.float32),
        pltpu.SemaphoreType.DMA,
    ],
)
def _deg_kernel(dst_hbm, w_hbm, zeros_hbm, deg_out, didx, wv, deg_sh, sem):
    c = lax.axis_index("c")
    s = lax.axis_index("s")
    wid = s * 2 + c
    pltpu.sync_copy(zeros_hbm.at[pl.ds(s * TSLAB, TSLAB)],
                    deg_sh.at[pl.ds(s * TSLAB, TSLAB)])
    pltpu.sync_copy(dst_hbm.at[pl.ds(wid * RPW, RPW)], didx)
    pltpu.sync_copy(w_hbm.at[pl.ds(wid * RPW, RPW)], wv)
    plsc.subcore_barrier()

    @pl.loop(0, RPW)
    def _(j):
        pltpu.async_copy(wv.at[j], deg_sh.at[didx.at[j]], sem, add=True)

    @pl.loop(0, RPW)
    def _(j):
        pltpu.make_async_copy(wv.at[j], deg_sh.at[didx.at[j]], sem).wait()

    plsc.subcore_barrier()
    pltpu.sync_copy(deg_sh.at[pl.ds(s * TSLAB, TSLAB)],
                    deg_out.at[c].at[pl.ds(s * TSLAB, TSLAB)])


@functools.partial(
    pl.kernel,
    mesh=_mesh,
    compiler_params=_sc_params,
    out_type=jax.ShapeDtypeStruct((2, NP, H), jnp.float32),
    scratch_types=[
        pltpu.VMEM((RPW, RW), jnp.int32),
        pltpu.VMEM((RPW, RW), jnp.int32),
        pltpu.VMEM((RPW, RW), jnp.float32),
        pltpu.VMEM((NP,), jnp.float32),
        pltpu.VMEM((RW,), jnp.float32),
        pltpu.VMEM((RW, H), jnp.float32),
        pltpu.VMEM((RW, H), jnp.float32),
        pltpu.VMEM((RW, H), jnp.float32),
        pltpu.VMEM((RW, H), jnp.float32),
        pltpu.VMEM_SHARED((N, H), jnp.float32),
        pltpu.VMEM_SHARED((NP, H), jnp.float32),
        pltpu.SemaphoreType.DMA,
        pltpu.SemaphoreType.DMA,
        pltpu.SemaphoreType.DMA,
        pltpu.SemaphoreType.DMA,
    ],
)
def _agg_kernel(h_hbm, dis_hbm, src_hbm, dst_hbm, w_hbm, zeros_hbm, out_hbm,
                sidx, didx, wv, dis_t, norm_b, rows0, rows1, msg0, msg1,
                h_sh, out_sh, gsem0, gsem1, ssem0, ssem1):
    c = lax.axis_index("c")
    s = lax.axis_index("s")
    wid = s * 2 + c
    pltpu.sync_copy(zeros_hbm.at[pl.ds(s * TSLAB, TSLAB)],
                    out_sh.at[pl.ds(s * TSLAB, TSLAB)])

    @pl.when(s < 15)
    def _():
        pltpu.sync_copy(h_hbm.at[pl.ds(s * TSLAB, TSLAB)],
                        h_sh.at[pl.ds(s * TSLAB, TSLAB)])

    @pl.when(s == 15)
    def _():
        pltpu.sync_copy(h_hbm.at[pl.ds(15 * TSLAB, N - 15 * TSLAB)],
                        h_sh.at[pl.ds(15 * TSLAB, N - 15 * TSLAB)])

    pltpu.sync_copy(src_hbm.at[pl.ds(wid * RPW, RPW)], sidx)
    pltpu.sync_copy(dst_hbm.at[pl.ds(wid * RPW, RPW)], didx)
    pltpu.sync_copy(w_hbm.at[pl.ds(wid * RPW, RPW)], wv)
    pltpu.sync_copy(dis_hbm, dis_t)
    plsc.subcore_barrier()

    lanes = jnp.arange(16, dtype=jnp.int32)
    rep = lanes // 8
    col = lanes - 8 * rep

    def compute_row(j, rows, msg):
        for v in range(RW // 16):
            s16 = sidx[j, pl.ds(16 * v, 16)]
            d16 = didx[j, pl.ds(16 * v, 16)]
            w16 = wv[j, pl.ds(16 * v, 16)]
            nrm = plsc.load_gather(dis_t, [s16]) * w16 * \
                plsc.load_gather(dis_t, [d16])
            norm_b[pl.ds(16 * v, 16)] = nrm

        @plsc.parallel_loop(0, RW // 2, unroll=8)
        def _(t):
            rowp = 2 * t + rep
            vals = plsc.load_gather(rows, [rowp, col])
            nrm16 = plsc.load_gather(norm_b, [rowp])
            plsc.store_scatter(msg, [rowp, col], vals * nrm16)

    def gather_wait(j, rows, gsem):
        pltpu.make_async_copy(h_sh.at[sidx.at[j]], rows, gsem).wait()

    def scatter_wait(j, msg, ssem):
        pltpu.make_async_copy(msg, out_sh.at[didx.at[j]], ssem).wait()

    pltpu.async_copy(h_sh.at[sidx.at[0]], rows0, gsem0)

    @pl.loop(0, RPW // 2)
    def _(p):
        j0 = 2 * p
        j1 = 2 * p + 1
        pltpu.async_copy(h_sh.at[sidx.at[j1]], rows1, gsem1)
        gather_wait(j0, rows0, gsem0)

        @pl.when(p > 0)
        def _():
            scatter_wait(j0 - 2, msg0, ssem0)

        compute_row(j0, rows0, msg0)
        pltpu.async_copy(msg0, out_sh.at[didx.at[j0]], ssem0, add=True)

        @pl.when(p < RPW // 2 - 1)
        def _():
            pltpu.async_copy(h_sh.at[sidx.at[j0 + 2]], rows0, gsem0)

        gather_wait(j1, rows1, gsem1)

        @pl.when(p > 0)
        def _():
            scatter_wait(j1 - 2, msg1, ssem1)

        compute_row(j1, rows1, msg1)
        pltpu.async_copy(msg1, out_sh.at[didx.at[j1]], ssem1, add=True)

    scatter_wait(RPW - 2, msg0, ssem0)
    scatter_wait(RPW - 1, msg1, ssem1)
    plsc.subcore_barrier()
    pltpu.sync_copy(out_sh.at[pl.ds(s * TSLAB, TSLAB)],
                    out_hbm.at[c].at[pl.ds(s * TSLAB, TSLAB)])


def _mm1_body(x_ref, w_ref, o_ref):
    o_ref[...] = jnp.dot(x_ref[...], w_ref[...],
                         preferred_element_type=jnp.float32)


def _dis_body(deg_ref, o_ref):
    o_ref[...] = lax.rsqrt(deg_ref[0] + deg_ref[1] + 1.0)


def _layer1_body(agg_ref, h_ref, dis_ref, b_ref, w_ref, o_ref):
    z = (agg_ref[0, :N] + agg_ref[1, :N]
         + h_ref[...] * dis_ref[:N] * dis_ref[:N] + b_ref[...])
    z = jnp.maximum(z, 0.0)
    o_ref[...] = jnp.dot(z, w_ref[...], preferred_element_type=jnp.float32)


def _layer2_body(agg_ref, h_ref, dis_ref, b_ref, o_ref):
    z = (agg_ref[0, :N] + agg_ref[1, :N]
         + h_ref[...] * dis_ref[:N] * dis_ref[:N] + b_ref[...])
    m = jnp.max(z, axis=1, keepdims=True)
    e = jnp.exp(z - m)
    o_ref[...] = (z - m) - jnp.log(jnp.sum(e, axis=1, keepdims=True))


def kernel(x, edge_index, edge_attr, W1, b1, W2, b2):
    ei = edge_index.astype(jnp.int32)
    src = jnp.pad(ei[0], (0, EP - E)).reshape(NROWS, RW)
    dst = jnp.pad(ei[1], (0, EP - E)).reshape(NROWS, RW)
    w2d = jnp.pad(edge_attr, (0, EP - E)).reshape(NROWS, RW)
    zeros_nh = jnp.zeros((NP, H), jnp.float32)
    zeros_n = jnp.zeros((NP,), jnp.float32)

    deg2 = _deg_kernel(dst, w2d, zeros_n)

    h1 = pl.pallas_call(
        _mm1_body,
        out_shape=jax.ShapeDtypeStruct((N, H), jnp.float32),
    )(x, W1)

    dis2d = pl.pallas_call(
        _dis_body,
        out_shape=jax.ShapeDtypeStruct((NP // 128, 128), jnp.float32),
    )(deg2.reshape(2, NP // 128, 128))
    dis = dis2d.reshape(NP)
    dis_col = dis2d.reshape(NP, 1)

    agg1 = _agg_kernel(h1, dis, src, dst, w2d, zeros_nh)

    h2 = pl.pallas_call(
        _layer1_body,
        out_shape=jax.ShapeDtypeStruct((N, H), jnp.float32),
    )(agg1, h1, dis_col, b1.reshape(1, H), W2)

    agg2 = _agg_kernel(h2, dis, src, dst, w2d, zeros_nh)

    out = pl.pallas_call(
        _layer2_body,
        out_shape=jax.ShapeDtypeStruct((N, H), jnp.float32),
    )(agg2, h2, dis_col, b2.reshape(1, H))
    return out


# speedup vs baseline: 1.3437x; 1.0146x over previous
import functools

import jax
import jax.numpy as jnp
from jax import lax
from jax.experimental import pallas as pl
from jax.experimental.pallas import tpu as pltpu
from jax.experimental.pallas import tpu_sc as plsc

N = 10000
E = 320000
D = 128
H = 8
NP = 10240
RW = 128
NWORK = 32
RPW = 80
NROWS = NWORK * RPW
EP = NROWS * RW
TSLAB = NP // 16

_mesh = plsc.VectorSubcoreMesh(core_axis_name="c", subcore_axis_name="s")
_sc_params = pltpu.CompilerParams(
    use_tc_tiling_on_sc=False, needs_layout_passes=False)


@functools.partial(
    pl.kernel,
    mesh=_mesh,
    compiler_params=_sc_params,
    out_type=jax.ShapeDtypeStruct((2, NP), jnp.float32),
    scratch_types=[
        pltpu.VMEM((RPW, RW), jnp.int32),
        pltpu.VMEM((RPW, RW), jnp.float32),
        pltpu.VMEM_SHARED((NP,), jnp.float32),
        pltpu.SemaphoreType.DMA,
    ],
)
def _deg_kernel(dst_hbm, w_hbm, zeros_hbm, deg_out, didx, wv, deg_sh, sem):
    c = lax.axis_index("c")
    s = lax.axis_index("s")
    wid = s * 2 + c
    pltpu.sync_copy(zeros_hbm.at[pl.ds(s * TSLAB, TSLAB)],
                    deg_sh.at[pl.ds(s * TSLAB, TSLAB)])
    pltpu.sync_copy(dst_hbm.at[pl.ds(wid * RPW, RPW)], didx)
    pltpu.sync_copy(w_hbm.at[pl.ds(wid * RPW, RPW)], wv)
    plsc.subcore_barrier()

    @pl.loop(0, RPW)
    def _(j):
        pltpu.async_copy(wv.at[j], deg_sh.at[didx.at[j]], sem, add=True)

    @pl.loop(0, RPW)
    def _(j):
        pltpu.make_async_copy(wv.at[j], deg_sh.at[didx.at[j]], sem).wait()

    plsc.subcore_barrier()
    pltpu.sync_copy(deg_sh.at[pl.ds(s * TSLAB, TSLAB)],
                    deg_out.at[c].at[pl.ds(s * TSLAB, TSLAB)])


@functools.partial(
    pl.kernel,
    mesh=_mesh,
    compiler_params=_sc_params,
    out_type=jax.ShapeDtypeStruct((2, NP, H), jnp.float32),
    scratch_types=[
        pltpu.VMEM((RPW, RW), jnp.int32),
        pltpu.VMEM((RPW, RW), jnp.int32),
        pltpu.VMEM((RPW, RW), jnp.float32),
        pltpu.VMEM((NP,), jnp.float32),
        pltpu.VMEM((RW,), jnp.float32),
        pltpu.VMEM((RW, H), jnp.float32),
        pltpu.VMEM((RW, H), jnp.float32),
        pltpu.VMEM((RW, H), jnp.float32),
        pltpu.VMEM((RW, H), jnp.float32),
        pltpu.VMEM_SHARED((N, H), jnp.float32),
        pltpu.VMEM_SHARED((NP, H), jnp.float32),
        pltpu.SemaphoreType.DMA,
        pltpu.SemaphoreType.DMA,
        pltpu.SemaphoreType.DMA,
        pltpu.SemaphoreType.DMA,
    ],
)
def _agg_kernel(h_hbm, dis_hbm, src_hbm, dst_hbm, w_hbm, zeros_hbm, out_hbm,
                sidx, didx, wv, dis_t, norm_b, rows0, rows1, msg0, msg1,
                h_sh, out_sh, gsem0, gsem1, ssem0, ssem1):
    c = lax.axis_index("c")
    s = lax.axis_index("s")
    wid = s * 2 + c
    pltpu.sync_copy(zeros_hbm.at[pl.ds(s * TSLAB, TSLAB)],
                    out_sh.at[pl.ds(s * TSLAB, TSLAB)])

    @pl.when(s < 15)
    def _():
        pltpu.sync_copy(h_hbm.at[pl.ds(s * TSLAB, TSLAB)],
                        h_sh.at[pl.ds(s * TSLAB, TSLAB)])

    @pl.when(s == 15)
    def _():
        pltpu.sync_copy(h_hbm.at[pl.ds(15 * TSLAB, N - 15 * TSLAB)],
                        h_sh.at[pl.ds(15 * TSLAB, N - 15 * TSLAB)])

    pltpu.sync_copy(src_hbm.at[pl.ds(wid * RPW, RPW)], sidx)
    pltpu.sync_copy(dst_hbm.at[pl.ds(wid * RPW, RPW)], didx)
    pltpu.sync_copy(w_hbm.at[pl.ds(wid * RPW, RPW)], wv)
    pltpu.sync_copy(dis_hbm, dis_t)
    plsc.subcore_barrier()

    lanes = jnp.arange(16, dtype=jnp.int32)
    rep = lanes // 8
    col = lanes - 8 * rep

    def compute_row(j, rows, msg):
        @plsc.parallel_loop(0, RW // 16, unroll=8)
        def _(v):
            s16 = sidx[j, pl.ds(16 * v, 16)]
            d16 = didx[j, pl.ds(16 * v, 16)]
            w16 = wv[j, pl.ds(16 * v, 16)]
            nrm = plsc.load_gather(dis_t, [s16]) * w16 * \
                plsc.load_gather(dis_t, [d16])
            norm_b[pl.ds(16 * v, 16)] = nrm

        @plsc.parallel_loop(0, RW // 2, unroll=8)
        def _(t):
            rowp = 2 * t + rep
            vals = plsc.load_gather(rows, [rowp, col])
            nrm16 = plsc.load_gather(norm_b, [rowp])
            plsc.store_scatter(msg, [rowp, col], vals * nrm16)

    def gather_wait(j, rows, gsem):
        pltpu.make_async_copy(h_sh.at[sidx.at[j]], rows, gsem).wait()

    def scatter_wait(j, msg, ssem):
        pltpu.make_async_copy(msg, out_sh.at[didx.at[j]], ssem).wait()

    pltpu.async_copy(h_sh.at[sidx.at[0]], rows0, gsem0)

    @pl.loop(0, RPW // 2)
    def _(p):
        j0 = 2 * p
        j1 = 2 * p + 1
        pltpu.async_copy(h_sh.at[sidx.at[j1]], rows1, gsem1)
        gather_wait(j0, rows0, gsem0)

        @pl.when(p > 0)
        def _():
            scatter_wait(j0 - 2, msg0, ssem0)

        compute_row(j0, rows0, msg0)
        pltpu.async_copy(msg0, out_sh.at[didx.at[j0]], ssem0, add=True)

        @pl.when(p < RPW // 2 - 1)
        def _():
            pltpu.async_copy(h_sh.at[sidx.at[j0 + 2]], rows0, gsem0)

        gather_wait(j1, rows1, gsem1)

        @pl.when(p > 0)
        def _():
            scatter_wait(j1 - 2, msg1, ssem1)

        compute_row(j1, rows1, msg1)
        pltpu.async_copy(msg1, out_sh.at[didx.at[j1]], ssem1, add=True)

    scatter_wait(RPW - 2, msg0, ssem0)
    scatter_wait(RPW - 1, msg1, ssem1)
    plsc.subcore_barrier()
    pltpu.sync_copy(out_sh.at[pl.ds(s * TSLAB, TSLAB)],
                    out_hbm.at[c].at[pl.ds(s * TSLAB, TSLAB)])


def _mm1_body(x_ref, w_ref, o_ref):
    o_ref[...] = jnp.dot(x_ref[...], w_ref[...],
                         preferred_element_type=jnp.float32)


def _dis_body(deg_ref, o_ref):
    o_ref[...] = lax.rsqrt(deg_ref[0] + deg_ref[1] + 1.0)


def _layer1_body(agg_ref, h_ref, dis_ref, b_ref, w_ref, o_ref):
    z = (agg_ref[0, :N] + agg_ref[1, :N]
         + h_ref[...] * dis_ref[:N] * dis_ref[:N] + b_ref[...])
    z = jnp.maximum(z, 0.0)
    o_ref[...] = jnp.dot(z, w_ref[...], preferred_element_type=jnp.float32)


def _layer2_body(agg_ref, h_ref, dis_ref, b_ref, o_ref):
    z = (agg_ref[0, :N] + agg_ref[1, :N]
         + h_ref[...] * dis_ref[:N] * dis_ref[:N] + b_ref[...])
    m = jnp.max(z, axis=1, keepdims=True)
    e = jnp.exp(z - m)
    o_ref[...] = (z - m) - jnp.log(jnp.sum(e, axis=1, keepdims=True))


def kernel(x, edge_index, edge_attr, W1, b1, W2, b2):
    ei = edge_index.astype(jnp.int32)
    src = jnp.pad(ei[0], (0, EP - E)).reshape(NROWS, RW)
    dst = jnp.pad(ei[1], (0, EP - E)).reshape(NROWS, RW)
    w2d = jnp.pad(edge_attr, (0, EP - E)).reshape(NROWS, RW)
    zeros_nh = jnp.zeros((NP, H), jnp.float32)
    zeros_n = jnp.zeros((NP,), jnp.float32)

    deg2 = _deg_kernel(dst, w2d, zeros_n)

    h1 = pl.pallas_call(
        _mm1_body,
        out_shape=jax.ShapeDtypeStruct((N, H), jnp.float32),
    )(x, W1)

    dis2d = pl.pallas_call(
        _dis_body,
        out_shape=jax.ShapeDtypeStruct((NP // 128, 128), jnp.float32),
    )(deg2.reshape(2, NP // 128, 128))
    dis = dis2d.reshape(NP)
    dis_col = dis2d.reshape(NP, 1)

    agg1 = _agg_kernel(h1, dis, src, dst, w2d, zeros_nh)

    h2 = pl.pallas_call(
        _layer1_body,
        out_shape=jax.ShapeDtypeStruct((N, H), jnp.float32),
    )(agg1, h1, dis_col, b1.reshape(1, H), W2)

    agg2 = _agg_kernel(h2, dis, src, dst, w2d, zeros_nh)

    out = pl.pallas_call(
        _layer2_body,
        out_shape=jax.ShapeDtypeStruct((N, H), jnp.float32),
    )(agg2, h2, dis_col, b2.reshape(1, H))
    return out
